# Initial kernel scaffold; baseline (speedup 1.0000x reference)
import jax
import jax.numpy as jnp
from jax.experimental import pallas as pl


def kernel(x, edge_index, edge_attr, task_features, params):
    raise NotImplementedError("write your pallas kernel here")


# speedup vs baseline: 3.4660x; 3.4660x over previous
import functools

import jax
import jax.numpy as jnp
from jax import lax
from jax.experimental import pallas as pl
from jax.experimental.pallas import tpu as pltpu
from jax.experimental.pallas import tpu_sc as plsc

N = 10000
E = 320000
DIN = 128
H = 64
OUT = 10
L = 3
W128 = 2 * H

NC = 2
NS = 16
NW = NC * NS
EPW = E // NW
C = 80
NCHP = 128
EPWP = NCHP * C
EPAD = NW * EPWP
ACCR = N + 16
NPS = 624
TAIL = N - NPS * NS
TOFF = NPS * NS

_PREC = lax.Precision.HIGHEST


def _lrelu(x, s):
    return jnp.where(x >= 0, x, s * x)


def _ln(x, g, b, eps=1e-5):
    m = jnp.mean(x, axis=-1, keepdims=True)
    v = jnp.mean(jnp.square(x - m), axis=-1, keepdims=True)
    return (x - m) * lax.rsqrt(v + eps) * g + b


def _encode_call(x, W, b, g, be):
    BN = 2000

    def body(x_ref, W_ref, b_ref, g_ref, be_ref, o_ref):
        h = jnp.dot(x_ref[...], W_ref[...], preferred_element_type=jnp.float32,
                    precision=_PREC) + b_ref[...]
        o_ref[...] = _lrelu(_ln(h, g_ref[...], be_ref[...]), 0.01)

    return pl.pallas_call(
        body,
        grid=(N // BN,),
        in_specs=[
            pl.BlockSpec((BN, DIN), lambda i: (i, 0)),
            pl.BlockSpec((DIN, H), lambda i: (0, 0)),
            pl.BlockSpec((H,), lambda i: (0,)),
            pl.BlockSpec((H,), lambda i: (0,)),
            pl.BlockSpec((H,), lambda i: (0,)),
        ],
        out_specs=pl.BlockSpec((BN, H), lambda i: (i, 0)),
        out_shape=jax.ShapeDtypeStruct((N, H), jnp.float32),
    )(x, W, b, g, be)


def _dis_call(degp):
    BN = 2000

    def body(d_ref, o_ref):
        t = d_ref[0] + d_ref[1]
        cnt = t[:, 0:1]
        o_ref[...] = lax.rsqrt(cnt + 1.0)

    return pl.pallas_call(
        body,
        grid=(N // BN,),
        in_specs=[pl.BlockSpec((NC, BN, W128), lambda i: (0, i, 0))],
        out_specs=pl.BlockSpec((BN, 1), lambda i: (i, 0)),
        out_shape=jax.ShapeDtypeStruct((N, 1), jnp.float32),
    )(degp)


def _layer_a_call(h, dis, Wcat, b1):
    BN = 2000

    def body(h_ref, dis_ref, W_ref, b1_ref, AG_ref, BB_ref):
        t = jnp.dot(h_ref[...], W_ref[...], preferred_element_type=jnp.float32,
                    precision=_PREC)
        d2 = dis_ref[...]
        AG_ref[...] = jnp.concatenate([t[:, :H], t[:, H:2 * H] * d2], axis=1)
        BB_ref[...] = jnp.concatenate(
            [t[:, 2 * H:] + b1_ref[...], jnp.zeros((BN, H), jnp.float32)],
            axis=1)

    return pl.pallas_call(
        body,
        grid=(N // BN,),
        in_specs=[
            pl.BlockSpec((BN, H), lambda i: (i, 0)),
            pl.BlockSpec((BN, 1), lambda i: (i, 0)),
            pl.BlockSpec((H, 3 * H), lambda i: (0, 0)),
            pl.BlockSpec((H,), lambda i: (0,)),
        ],
        out_specs=[
            pl.BlockSpec((BN, W128), lambda i: (i, 0)),
            pl.BlockSpec((BN, W128), lambda i: (i, 0)),
        ],
        out_shape=[
            jax.ShapeDtypeStruct((N, W128), jnp.float32),
            jax.ShapeDtypeStruct((N, W128), jnp.float32),
        ],
    )(h, dis, Wcat, b1)


def _edge_mlp_call(pA, pB, ea, W2, b2, g1, be1, g2, be2):
    BE = 2048

    def body(pA_ref, pB_ref, ea_ref, W2_ref, b2_ref, g1_ref, be1_ref, g2_ref,
             be2_ref, o_ref):
        i = pl.program_id(0)
        a = pA_ref[...]
        pre = a[:, :H] + pB_ref[:, :H]
        ef1 = _lrelu(_ln(pre, g1_ref[...], be1_ref[...]), 0.2)
        z = jnp.dot(ef1, W2_ref[...], preferred_element_type=jnp.float32,
                    precision=_PREC) + b2_ref[...]
        ef2 = _lrelu(_ln(z, g2_ref[...], be2_ref[...]), 0.2)
        ef2 = ef2 * jnp.reshape(ea_ref[pl.ds(i * BE, BE)], (BE, 1))
        o_ref[...] = jnp.concatenate([ef2, a[:, H:]], axis=1)

    return pl.pallas_call(
        body,
        grid=(EPAD // BE,),
        in_specs=[
            pl.BlockSpec((BE, W128), lambda i: (i, 0)),
            pl.BlockSpec((BE, W128), lambda i: (i, 0)),
            pl.BlockSpec((EPAD,), lambda i: (0,)),
            pl.BlockSpec((H, H), lambda i: (0, 0)),
            pl.BlockSpec((H,), lambda i: (0,)),
            pl.BlockSpec((H,), lambda i: (0,)),
            pl.BlockSpec((H,), lambda i: (0,)),
            pl.BlockSpec((H,), lambda i: (0,)),
            pl.BlockSpec((H,), lambda i: (0,)),
        ],
        out_specs=pl.BlockSpec((BE, W128), lambda i: (i, 0)),
        out_shape=jax.ShapeDtypeStruct((EPAD, W128), jnp.float32),
    )(pA, pB, ea, W2, b2, g1, be1, g2, be2)


def _combine_call(Sp, AG, dis, bg, ng, nbe, h0):
    BN = 2000
    residual = h0 is not None

    def body(sp_ref, ag_ref, dis_ref, bg_ref, ng_ref, nbe_ref, *rest):
        if residual:
            h0_ref, o_ref = rest
        else:
            (o_ref,) = rest
        t = sp_ref[0] + sp_ref[1]
        d2 = dis_ref[...]
        x1 = (t[:, H:] + ag_ref[:, H:]) * d2 + bg_ref[...]
        h = _lrelu(_ln(x1 + t[:, :H], ng_ref[...], nbe_ref[...]), 0.01)
        if residual:
            h = h + h0_ref[...]
        o_ref[...] = h

    in_specs = [
        pl.BlockSpec((NC, BN, W128), lambda i: (0, i, 0)),
        pl.BlockSpec((BN, W128), lambda i: (i, 0)),
        pl.BlockSpec((BN, 1), lambda i: (i, 0)),
        pl.BlockSpec((H,), lambda i: (0,)),
        pl.BlockSpec((H,), lambda i: (0,)),
        pl.BlockSpec((H,), lambda i: (0,)),
    ]
    args = [Sp, AG, dis, bg, ng, nbe]
    if residual:
        in_specs.append(pl.BlockSpec((BN, H), lambda i: (i, 0)))
        args.append(h0)
    return pl.pallas_call(
        body,
        grid=(N // BN,),
        in_specs=in_specs,
        out_specs=pl.BlockSpec((BN, H), lambda i: (i, 0)),
        out_shape=jax.ShapeDtypeStruct((N, H), jnp.float32),
    )(*args)


def _readout_call(h, tf2d, p):
    def body(h_ref, tf_ref, p1W, p1b, p2W, p2b, tW, tb, tg, tbe,
             v1W, v1b, vg, vbe, v2W, v2b, a1W, a1b, ag, abe, a2W, a2b,
             o_ref):
        hh = h_ref[...]
        w = _lrelu(jnp.dot(hh, p1W[...], preferred_element_type=jnp.float32,
                           precision=_PREC) + p1b[...], 0.01)
        s = jnp.dot(w, p2W[...], preferred_element_type=jnp.float32,
                    precision=_PREC) + p2b[...]
        m = jnp.max(s)
        e = jnp.exp(s - m)
        wn = e / jnp.sum(e)
        gf = lax.dot_general(wn, hh, (((0,), (0,)), ((), ())),
                             preferred_element_type=jnp.float32,
                             precision=_PREC)
        te = _lrelu(_ln(jnp.dot(tf_ref[...], tW[...],
                                preferred_element_type=jnp.float32,
                                precision=_PREC) + tb[...],
                        tg[...], tbe[...]), 0.01)
        cf = jnp.concatenate([gf, te], axis=1)
        v = jnp.dot(_lrelu(_ln(jnp.dot(cf, v1W[...],
                                       preferred_element_type=jnp.float32,
                                       precision=_PREC) + v1b[...],
                               vg[...], vbe[...]), 0.01), v2W[...],
                    preferred_element_type=jnp.float32,
                    precision=_PREC) + v2b[...]
        a = jnp.dot(_lrelu(_ln(jnp.dot(cf, a1W[...],
                                       preferred_element_type=jnp.float32,
                                       precision=_PREC) + a1b[...],
                               ag[...], abe[...]), 0.01), a2W[...],
                    preferred_element_type=jnp.float32,
                    precision=_PREC) + a2b[...]
        o_ref[...] = v + (a - jnp.mean(a, axis=-1, keepdims=True))

    return pl.pallas_call(
        body,
        out_shape=jax.ShapeDtypeStruct((1, OUT), jnp.float32),
    )(h, tf2d, p["p1_W"], p["p1_b"], p["p2_W"], p["p2_b"],
      p["t_W"], p["t_b"], p["t_g"], p["t_be"],
      p["v1_W"], p["v1_b"], p["v_g"], p["v_be"], p["v2_W"], p["v2_b"],
      p["a1_W"], p["a1_b"], p["a_g"], p["a_be"], p["a2_W"], p["a2_b"])


def _sc_mesh():
    return plsc.VectorSubcoreMesh(core_axis_name="c", subcore_axis_name="s")


def _zero_acc(z_hbm, acc, s):
    pltpu.sync_copy(z_hbm, acc.at[pl.ds(s * NPS, NPS)])

    @pl.when(s == NS - 1)
    def _():
        pltpu.sync_copy(z_hbm.at[pl.ds(0, TAIL)], acc.at[pl.ds(TOFF, TAIL)])


def _dump_acc(acc, out_hbm, c, s):
    pltpu.sync_copy(acc.at[pl.ds(s * NPS, NPS)],
                    out_hbm.at[c, pl.ds(s * NPS, NPS)])

    @pl.when(s == NS - 1)
    def _():
        pltpu.sync_copy(acc.at[pl.ds(TOFF, TAIL)],
                        out_hbm.at[c, pl.ds(TOFF, TAIL)])


def _sc_deg_call(dsts1d, ones128, z128):
    @functools.partial(
        pl.kernel,
        mesh=_sc_mesh(),
        out_type=jax.ShapeDtypeStruct((NC, N, W128), jnp.float32),
        scratch_types=[
            pltpu.VMEM((C,), jnp.int32),
            pltpu.VMEM((C, W128), jnp.float32),
            pltpu.VMEM_SHARED((ACCR, W128), jnp.float32),
        ],
    )
    def k(dst_hbm, ones_hbm, z_hbm, out_hbm, idx1, ones_v, acc):
        c = lax.axis_index("c")
        s = lax.axis_index("s")
        wid = s * NC + c
        pltpu.sync_copy(ones_hbm, ones_v)
        _zero_acc(z_hbm, acc, s)
        plsc.subcore_barrier()

        def chunk(j, carry):
            pltpu.sync_copy(dst_hbm.at[pl.ds(wid * EPWP + j * C, C)], idx1)
            pltpu.sync_copy(ones_v, acc.at[idx1], add=True)
            return carry

        lax.fori_loop(0, NCHP, chunk, 0)
        plsc.subcore_barrier()
        _dump_acc(acc, out_hbm, c, s)

    return k(dsts1d, ones128, z128)


def _sc_gather(AG, BB, srcg1d, dstg1d):
    @functools.partial(
        pl.kernel,
        mesh=_sc_mesh(),
        out_type=[
            jax.ShapeDtypeStruct((EPAD, W128), jnp.float32),
            jax.ShapeDtypeStruct((EPAD, W128), jnp.float32),
        ],
        scratch_types=[
            pltpu.VMEM((C,), jnp.int32),
            pltpu.VMEM((C,), jnp.int32),
            pltpu.VMEM((C, W128), jnp.float32),
            pltpu.VMEM((C, W128), jnp.float32),
            pltpu.SemaphoreType.DMA,
        ],
    )
    def k(AG_hbm, BB_hbm, src_hbm, dstg_hbm, pA_hbm, pB_hbm,
          idx_s, idx_dg, bufA, bufB, sem):
        c = lax.axis_index("c")
        s = lax.axis_index("s")
        wid = s * NC + c

        def chunk(j, carry):
            base = wid * EPWP + j * C
            pltpu.sync_copy(src_hbm.at[pl.ds(base, C)], idx_s)
            pltpu.sync_copy(dstg_hbm.at[pl.ds(base, C)], idx_dg)
            pltpu.async_copy(AG_hbm.at[idx_s], bufA, sem).wait()
            pltpu.async_copy(BB_hbm.at[idx_dg], bufB, sem).wait()
            pltpu.sync_copy(bufA, pA_hbm.at[pl.ds(base, C)])
            pltpu.sync_copy(bufB, pB_hbm.at[pl.ds(base, C)])
            return carry

        lax.fori_loop(0, NCHP, chunk, 0)

    return k(AG, BB, srcg1d, dstg1d)


def _sc_scatter(msg2, dsts1d, z128):
    @functools.partial(
        pl.kernel,
        mesh=_sc_mesh(),
        out_type=jax.ShapeDtypeStruct((NC, N, W128), jnp.float32),
        scratch_types=[
            pltpu.VMEM((C,), jnp.int32),
            pltpu.VMEM((C, W128), jnp.float32),
            pltpu.VMEM_SHARED((ACCR, W128), jnp.float32),
        ],
    )
    def k(ef_hbm, dst_hbm, z_hbm, s2_hbm, idx_d, bufE, acc):
        c = lax.axis_index("c")
        s = lax.axis_index("s")
        wid = s * NC + c
        _zero_acc(z_hbm, acc, s)
        plsc.subcore_barrier()

        def chunk(j, carry):
            base = wid * EPWP + j * C
            pltpu.sync_copy(dst_hbm.at[pl.ds(base, C)], idx_d)
            pltpu.sync_copy(ef_hbm.at[pl.ds(base, C)], bufE)
            pltpu.sync_copy(bufE, acc.at[idx_d], add=True)
            return carry

        lax.fori_loop(0, NCHP, chunk, 0)
        plsc.subcore_barrier()
        _dump_acc(acc, s2_hbm, c, s)

    return k(msg2, dsts1d, z128)


def _pad_per_worker(v, fill):
    vw = v.reshape(NW, EPW)
    return jnp.pad(vw, ((0, 0), (0, EPWP - EPW)),
                   constant_values=fill).reshape(-1)


def kernel(x, edge_index, edge_attr, task_features, params):
    p = params
    src = edge_index[0]
    dst = edge_index[1]
    srcg1d = _pad_per_worker(src, 0)
    dstg1d = _pad_per_worker(dst, 0)
    dsts1d = _pad_per_worker(dst, N)
    ea_p = _pad_per_worker(edge_attr, 0.0)
    z128 = jnp.zeros((NPS, W128), jnp.float32)
    ones128 = jnp.ones((C, W128), jnp.float32)

    degp = _sc_deg_call(dsts1d, ones128, z128)
    dis = _dis_call(degp)
    h = _encode_call(x, p["ne_W"], p["ne_b"], p["ne_g"], p["ne_be"])
    h0 = h
    for i in range(L):
        W1 = p["e1_W%d" % i]
        Wcat = jnp.concatenate([W1[:H], p["gcn_W%d" % i], W1[H:]], axis=1)
        AG, BB = _layer_a_call(h, dis, Wcat, p["e1_b%d" % i])
        pA, pB = _sc_gather(AG, BB, srcg1d, dstg1d)
        msg2 = _edge_mlp_call(pA, pB, ea_p, p["e2_W%d" % i], p["e2_b%d" % i],
                              p["e1_g%d" % i], p["e1_be%d" % i],
                              p["e2_g%d" % i], p["e2_be%d" % i])
        Sp = _sc_scatter(msg2, dsts1d, z128)
        h = _combine_call(Sp, AG, dis, p["gcn_b%d" % i],
                          p["n_g%d" % i], p["n_be%d" % i],
                          h0 if i > 0 else None)
        if i > 0:
            h0 = h
    out = _readout_call(h, task_features.reshape(1, 4), p)
    return out[0]


# speedup vs baseline: 3.7830x; 1.0915x over previous
import functools

import jax
import jax.numpy as jnp
from jax import lax
from jax.experimental import pallas as pl
from jax.experimental.pallas import tpu as pltpu
from jax.experimental.pallas import tpu_sc as plsc

N = 10000
E = 320000
DIN = 128
H = 64
OUT = 10
L = 3
W128 = 2 * H

NC = 2
NS = 16
NW = NC * NS
EPW = E // NW
C = 128
NCHP = 80
EPWP = NCHP * C
EPAD = NW * EPWP
ACCR = N + 16
NPS = 624
TAIL = N - NPS * NS
TOFF = NPS * NS

_PREC = lax.Precision.HIGHEST


def _lrelu(x, s):
    return jnp.where(x >= 0, x, s * x)


def _ln(x, g, b, eps=1e-5):
    m = jnp.mean(x, axis=-1, keepdims=True)
    v = jnp.mean(jnp.square(x - m), axis=-1, keepdims=True)
    return (x - m) * lax.rsqrt(v + eps) * g + b


def _encode_call(x, W, b, g, be):
    BN = 2000

    def body(x_ref, W_ref, b_ref, g_ref, be_ref, o_ref):
        h = jnp.dot(x_ref[...], W_ref[...], preferred_element_type=jnp.float32,
                    precision=_PREC) + b_ref[...]
        o_ref[...] = _lrelu(_ln(h, g_ref[...], be_ref[...]), 0.01)

    return pl.pallas_call(
        body,
        grid=(N // BN,),
        in_specs=[
            pl.BlockSpec((BN, DIN), lambda i: (i, 0)),
            pl.BlockSpec((DIN, H), lambda i: (0, 0)),
            pl.BlockSpec((H,), lambda i: (0,)),
            pl.BlockSpec((H,), lambda i: (0,)),
            pl.BlockSpec((H,), lambda i: (0,)),
        ],
        out_specs=pl.BlockSpec((BN, H), lambda i: (i, 0)),
        out_shape=jax.ShapeDtypeStruct((N, H), jnp.float32),
    )(x, W, b, g, be)


def _dis_call(degp):
    BN = 2000

    def body(d_ref, o_ref):
        t = d_ref[0] + d_ref[1]
        cnt = t[:, 0:1]
        o_ref[...] = lax.rsqrt(cnt + 1.0)

    return pl.pallas_call(
        body,
        grid=(N // BN,),
        in_specs=[pl.BlockSpec((NC, BN, W128), lambda i: (0, i, 0))],
        out_specs=pl.BlockSpec((BN, 1), lambda i: (i, 0)),
        out_shape=jax.ShapeDtypeStruct((N, 1), jnp.float32),
    )(degp)


def _layer_a_call(h, dis, Wcat, b1):
    BN = 2000

    def body(h_ref, dis_ref, W_ref, b1_ref, AG_ref, BB_ref):
        t = jnp.dot(h_ref[...], W_ref[...], preferred_element_type=jnp.float32,
                    precision=_PREC)
        d2 = dis_ref[...]
        AG_ref[...] = jnp.concatenate([t[:, :H], t[:, H:2 * H] * d2], axis=1)
        BB_ref[...] = jnp.concatenate(
            [t[:, 2 * H:] + b1_ref[...], jnp.zeros((BN, H), jnp.float32)],
            axis=1)

    return pl.pallas_call(
        body,
        grid=(N // BN,),
        in_specs=[
            pl.BlockSpec((BN, H), lambda i: (i, 0)),
            pl.BlockSpec((BN, 1), lambda i: (i, 0)),
            pl.BlockSpec((H, 3 * H), lambda i: (0, 0)),
            pl.BlockSpec((H,), lambda i: (0,)),
        ],
        out_specs=[
            pl.BlockSpec((BN, W128), lambda i: (i, 0)),
            pl.BlockSpec((BN, W128), lambda i: (i, 0)),
        ],
        out_shape=[
            jax.ShapeDtypeStruct((N, W128), jnp.float32),
            jax.ShapeDtypeStruct((N, W128), jnp.float32),
        ],
    )(h, dis, Wcat, b1)


def _edge_mlp_call(pA, pB, ea, W2, b2, g1, be1, g2, be2):
    BE = 2048

    def body(pA_ref, pB_ref, ea_ref, W2_ref, b2_ref, g1_ref, be1_ref, g2_ref,
             be2_ref, o_ref):
        i = pl.program_id(0)
        a = pA_ref[...]
        pre = a[:, :H] + pB_ref[:, :H]
        ef1 = _lrelu(_ln(pre, g1_ref[...], be1_ref[...]), 0.2)
        z = jnp.dot(ef1, W2_ref[...], preferred_element_type=jnp.float32,
                    precision=_PREC) + b2_ref[...]
        ef2 = _lrelu(_ln(z, g2_ref[...], be2_ref[...]), 0.2)
        ef2 = ef2 * jnp.reshape(ea_ref[pl.ds(i * BE, BE)], (BE, 1))
        o_ref[...] = jnp.concatenate([ef2, a[:, H:]], axis=1)

    return pl.pallas_call(
        body,
        grid=(EPAD // BE,),
        in_specs=[
            pl.BlockSpec((BE, W128), lambda i: (i, 0)),
            pl.BlockSpec((BE, W128), lambda i: (i, 0)),
            pl.BlockSpec((EPAD,), lambda i: (0,)),
            pl.BlockSpec((H, H), lambda i: (0, 0)),
            pl.BlockSpec((H,), lambda i: (0,)),
            pl.BlockSpec((H,), lambda i: (0,)),
            pl.BlockSpec((H,), lambda i: (0,)),
            pl.BlockSpec((H,), lambda i: (0,)),
            pl.BlockSpec((H,), lambda i: (0,)),
        ],
        out_specs=pl.BlockSpec((BE, W128), lambda i: (i, 0)),
        out_shape=jax.ShapeDtypeStruct((EPAD, W128), jnp.float32),
    )(pA, pB, ea, W2, b2, g1, be1, g2, be2)


def _combine_call(Sp, AG, dis, bg, ng, nbe, h0):
    BN = 2000
    residual = h0 is not None

    def body(sp_ref, ag_ref, dis_ref, bg_ref, ng_ref, nbe_ref, *rest):
        if residual:
            h0_ref, o_ref = rest
        else:
            (o_ref,) = rest
        t = sp_ref[0] + sp_ref[1]
        d2 = dis_ref[...]
        x1 = (t[:, H:] + ag_ref[:, H:]) * d2 + bg_ref[...]
        h = _lrelu(_ln(x1 + t[:, :H], ng_ref[...], nbe_ref[...]), 0.01)
        if residual:
            h = h + h0_ref[...]
        o_ref[...] = h

    in_specs = [
        pl.BlockSpec((NC, BN, W128), lambda i: (0, i, 0)),
        pl.BlockSpec((BN, W128), lambda i: (i, 0)),
        pl.BlockSpec((BN, 1), lambda i: (i, 0)),
        pl.BlockSpec((H,), lambda i: (0,)),
        pl.BlockSpec((H,), lambda i: (0,)),
        pl.BlockSpec((H,), lambda i: (0,)),
    ]
    args = [Sp, AG, dis, bg, ng, nbe]
    if residual:
        in_specs.append(pl.BlockSpec((BN, H), lambda i: (i, 0)))
        args.append(h0)
    return pl.pallas_call(
        body,
        grid=(N // BN,),
        in_specs=in_specs,
        out_specs=pl.BlockSpec((BN, H), lambda i: (i, 0)),
        out_shape=jax.ShapeDtypeStruct((N, H), jnp.float32),
    )(*args)


def _readout_call(h, tf2d, p):
    def body(h_ref, tf_ref, p1W, p1b, p2W, p2b, tW, tb, tg, tbe,
             v1W, v1b, vg, vbe, v2W, v2b, a1W, a1b, ag, abe, a2W, a2b,
             o_ref):
        hh = h_ref[...]
        w = _lrelu(jnp.dot(hh, p1W[...], preferred_element_type=jnp.float32,
                           precision=_PREC) + p1b[...], 0.01)
        s = jnp.dot(w, p2W[...], preferred_element_type=jnp.float32,
                    precision=_PREC) + p2b[...]
        m = jnp.max(s)
        e = jnp.exp(s - m)
        wn = e / jnp.sum(e)
        gf = lax.dot_general(wn, hh, (((0,), (0,)), ((), ())),
                             preferred_element_type=jnp.float32,
                             precision=_PREC)
        te = _lrelu(_ln(jnp.dot(tf_ref[...], tW[...],
                                preferred_element_type=jnp.float32,
                                precision=_PREC) + tb[...],
                        tg[...], tbe[...]), 0.01)
        cf = jnp.concatenate([gf, te], axis=1)
        v = jnp.dot(_lrelu(_ln(jnp.dot(cf, v1W[...],
                                       preferred_element_type=jnp.float32,
                                       precision=_PREC) + v1b[...],
                               vg[...], vbe[...]), 0.01), v2W[...],
                    preferred_element_type=jnp.float32,
                    precision=_PREC) + v2b[...]
        a = jnp.dot(_lrelu(_ln(jnp.dot(cf, a1W[...],
                                       preferred_element_type=jnp.float32,
                                       precision=_PREC) + a1b[...],
                               ag[...], abe[...]), 0.01), a2W[...],
                    preferred_element_type=jnp.float32,
                    precision=_PREC) + a2b[...]
        o_ref[...] = v + (a - jnp.mean(a, axis=-1, keepdims=True))

    return pl.pallas_call(
        body,
        out_shape=jax.ShapeDtypeStruct((1, OUT), jnp.float32),
    )(h, tf2d, p["p1_W"], p["p1_b"], p["p2_W"], p["p2_b"],
      p["t_W"], p["t_b"], p["t_g"], p["t_be"],
      p["v1_W"], p["v1_b"], p["v_g"], p["v_be"], p["v2_W"], p["v2_b"],
      p["a1_W"], p["a1_b"], p["a_g"], p["a_be"], p["a2_W"], p["a2_b"])


def _sc_mesh():
    return plsc.VectorSubcoreMesh(core_axis_name="c", subcore_axis_name="s")


def _zero_acc(z_hbm, acc, s):
    pltpu.sync_copy(z_hbm, acc.at[pl.ds(s * NPS, NPS)])

    @pl.when(s == NS - 1)
    def _():
        pltpu.sync_copy(z_hbm.at[pl.ds(0, TAIL)], acc.at[pl.ds(TOFF, TAIL)])


def _dump_acc(acc, out_hbm, c, s):
    pltpu.sync_copy(acc.at[pl.ds(s * NPS, NPS)],
                    out_hbm.at[c, pl.ds(s * NPS, NPS)])

    @pl.when(s == NS - 1)
    def _():
        pltpu.sync_copy(acc.at[pl.ds(TOFF, TAIL)],
                        out_hbm.at[c, pl.ds(TOFF, TAIL)])


def _sc_deg_call(dsts1d, ones128, z128):
    @functools.partial(
        pl.kernel,
        mesh=_sc_mesh(),
        out_type=jax.ShapeDtypeStruct((NC, N, W128), jnp.float32),
        scratch_types=[
            pltpu.VMEM((C,), jnp.int32),
            pltpu.VMEM((C, W128), jnp.float32),
            pltpu.VMEM_SHARED((ACCR, W128), jnp.float32),
        ],
    )
    def k(dst_hbm, ones_hbm, z_hbm, out_hbm, idx1, ones_v, acc):
        c = lax.axis_index("c")
        s = lax.axis_index("s")
        wid = s * NC + c
        pltpu.sync_copy(ones_hbm, ones_v)
        _zero_acc(z_hbm, acc, s)
        plsc.subcore_barrier()

        def chunk(j, carry):
            pltpu.sync_copy(dst_hbm.at[pl.ds(wid * EPWP + j * C, C)], idx1)
            pltpu.sync_copy(ones_v, acc.at[idx1], add=True)
            return carry

        lax.fori_loop(0, NCHP, chunk, 0)
        plsc.subcore_barrier()
        _dump_acc(acc, out_hbm, c, s)

    return k(dsts1d, ones128, z128)


def _sc_gather(AG, BB, srcg1d, dstg1d):
    @functools.partial(
        pl.kernel,
        mesh=_sc_mesh(),
        out_type=[
            jax.ShapeDtypeStruct((EPAD, W128), jnp.float32),
            jax.ShapeDtypeStruct((EPAD, W128), jnp.float32),
        ],
        scratch_types=[
            pltpu.VMEM((C,), jnp.int32),
            pltpu.VMEM((C,), jnp.int32),
            pltpu.VMEM((C, W128), jnp.float32),
            pltpu.VMEM((C, W128), jnp.float32),
            pltpu.SemaphoreType.DMA,
        ],
    )
    def k(AG_hbm, BB_hbm, src_hbm, dstg_hbm, pA_hbm, pB_hbm,
          idx_s, idx_dg, bufA, bufB, sem):
        c = lax.axis_index("c")
        s = lax.axis_index("s")
        wid = s * NC + c

        def chunk(j, carry):
            base = wid * EPWP + j * C
            pltpu.sync_copy(src_hbm.at[pl.ds(base, C)], idx_s)
            pltpu.sync_copy(dstg_hbm.at[pl.ds(base, C)], idx_dg)
            pltpu.async_copy(AG_hbm.at[idx_s], bufA, sem).wait()
            pltpu.async_copy(BB_hbm.at[idx_dg], bufB, sem).wait()
            pltpu.sync_copy(bufA, pA_hbm.at[pl.ds(base, C)])
            pltpu.sync_copy(bufB, pB_hbm.at[pl.ds(base, C)])
            return carry

        lax.fori_loop(0, NCHP, chunk, 0)

    return k(AG, BB, srcg1d, dstg1d)


def _sc_scatter(msg2, dsts1d, z128):
    @functools.partial(
        pl.kernel,
        mesh=_sc_mesh(),
        out_type=jax.ShapeDtypeStruct((NC, N, W128), jnp.float32),
        scratch_types=[
            pltpu.VMEM((C,), jnp.int32),
            pltpu.VMEM((C, W128), jnp.float32),
            pltpu.VMEM_SHARED((ACCR, W128), jnp.float32),
        ],
    )
    def k(ef_hbm, dst_hbm, z_hbm, s2_hbm, idx_d, bufE, acc):
        c = lax.axis_index("c")
        s = lax.axis_index("s")
        wid = s * NC + c
        _zero_acc(z_hbm, acc, s)
        plsc.subcore_barrier()

        def chunk(j, carry):
            base = wid * EPWP + j * C
            pltpu.sync_copy(dst_hbm.at[pl.ds(base, C)], idx_d)
            pltpu.sync_copy(ef_hbm.at[pl.ds(base, C)], bufE)
            pltpu.sync_copy(bufE, acc.at[idx_d], add=True)
            return carry

        lax.fori_loop(0, NCHP, chunk, 0)
        plsc.subcore_barrier()
        _dump_acc(acc, s2_hbm, c, s)

    return k(msg2, dsts1d, z128)


def _pad_per_worker(v, fill):
    vw = v.reshape(NW, EPW)
    return jnp.pad(vw, ((0, 0), (0, EPWP - EPW)),
                   constant_values=fill).reshape(-1)


def kernel(x, edge_index, edge_attr, task_features, params):
    p = params
    src = edge_index[0]
    dst = edge_index[1]
    srcg1d = _pad_per_worker(src, 0)
    dstg1d = _pad_per_worker(dst, 0)
    dsts1d = _pad_per_worker(dst, N)
    ea_p = _pad_per_worker(edge_attr, 0.0)
    z128 = jnp.zeros((NPS, W128), jnp.float32)
    ones128 = jnp.ones((C, W128), jnp.float32)

    degp = _sc_deg_call(dsts1d, ones128, z128)
    dis = _dis_call(degp)
    h = _encode_call(x, p["ne_W"], p["ne_b"], p["ne_g"], p["ne_be"])
    h0 = h
    for i in range(L):
        W1 = p["e1_W%d" % i]
        Wcat = jnp.concatenate([W1[:H], p["gcn_W%d" % i], W1[H:]], axis=1)
        AG, BB = _layer_a_call(h, dis, Wcat, p["e1_b%d" % i])
        pA, pB = _sc_gather(AG, BB, srcg1d, dstg1d)
        msg2 = _edge_mlp_call(pA, pB, ea_p, p["e2_W%d" % i], p["e2_b%d" % i],
                              p["e1_g%d" % i], p["e1_be%d" % i],
                              p["e2_g%d" % i], p["e2_be%d" % i])
        Sp = _sc_scatter(msg2, dsts1d, z128)
        h = _combine_call(Sp, AG, dis, p["gcn_b%d" % i],
                          p["n_g%d" % i], p["n_be%d" % i],
                          h0 if i > 0 else None)
        if i > 0:
            h0 = h
    out = _readout_call(h, task_features.reshape(1, 4), p)
    return out[0]


# speedup vs baseline: 4.6134x; 1.2195x over previous
import functools

import jax
import jax.numpy as jnp
from jax import lax
from jax.experimental import pallas as pl
from jax.experimental.pallas import tpu as pltpu
from jax.experimental.pallas import tpu_sc as plsc

N = 10000
E = 320000
DIN = 128
H = 64
OUT = 10
L = 3
W128 = 2 * H

NC = 2
NS = 16
NW = NC * NS
EPW = E // NW
C = 128
NCHP = 80
EPWP = NCHP * C
EPAD = NW * EPWP
ACCR = N + 16
NPS = 624
TAIL = N - NPS * NS
TOFF = NPS * NS

_PREC = lax.Precision.HIGHEST


def _lrelu(x, s):
    return jnp.where(x >= 0, x, s * x)


def _ln(x, g, b, eps=1e-5):
    m = jnp.mean(x, axis=-1, keepdims=True)
    v = jnp.mean(jnp.square(x - m), axis=-1, keepdims=True)
    return (x - m) * lax.rsqrt(v + eps) * g + b


def _encode_call(x, W, b, g, be):
    BN = 2000

    def body(x_ref, W_ref, b_ref, g_ref, be_ref, o_ref):
        h = jnp.dot(x_ref[...], W_ref[...], preferred_element_type=jnp.float32,
                    precision=_PREC) + b_ref[...]
        o_ref[...] = _lrelu(_ln(h, g_ref[...], be_ref[...]), 0.01)

    return pl.pallas_call(
        body,
        grid=(N // BN,),
        in_specs=[
            pl.BlockSpec((BN, DIN), lambda i: (i, 0)),
            pl.BlockSpec((DIN, H), lambda i: (0, 0)),
            pl.BlockSpec((H,), lambda i: (0,)),
            pl.BlockSpec((H,), lambda i: (0,)),
            pl.BlockSpec((H,), lambda i: (0,)),
        ],
        out_specs=pl.BlockSpec((BN, H), lambda i: (i, 0)),
        out_shape=jax.ShapeDtypeStruct((N, H), jnp.float32),
    )(x, W, b, g, be)


def _dis_call(degp):
    BN = 2000

    def body(d_ref, o_ref):
        t = d_ref[0] + d_ref[1]
        cnt = t[:, 0:1]
        o_ref[...] = lax.rsqrt(cnt + 1.0)

    return pl.pallas_call(
        body,
        grid=(N // BN,),
        in_specs=[pl.BlockSpec((NC, BN, W128), lambda i: (0, i, 0))],
        out_specs=pl.BlockSpec((BN, 1), lambda i: (i, 0)),
        out_shape=jax.ShapeDtypeStruct((N, 1), jnp.float32),
    )(degp)


def _layer_a_call(h, dis, Wcat, b1):
    BN = 2000

    def body(h_ref, dis_ref, W_ref, b1_ref, AG_ref, BB_ref):
        t = jnp.dot(h_ref[...], W_ref[...], preferred_element_type=jnp.float32,
                    precision=_PREC)
        d2 = dis_ref[...]
        AG_ref[...] = jnp.concatenate([t[:, :H], t[:, H:2 * H] * d2], axis=1)
        BB_ref[...] = jnp.concatenate(
            [t[:, 2 * H:] + b1_ref[...], jnp.zeros((BN, H), jnp.float32)],
            axis=1)

    return pl.pallas_call(
        body,
        grid=(N // BN,),
        in_specs=[
            pl.BlockSpec((BN, H), lambda i: (i, 0)),
            pl.BlockSpec((BN, 1), lambda i: (i, 0)),
            pl.BlockSpec((H, 3 * H), lambda i: (0, 0)),
            pl.BlockSpec((H,), lambda i: (0,)),
        ],
        out_specs=[
            pl.BlockSpec((BN, W128), lambda i: (i, 0)),
            pl.BlockSpec((BN, W128), lambda i: (i, 0)),
        ],
        out_shape=[
            jax.ShapeDtypeStruct((N, W128), jnp.float32),
            jax.ShapeDtypeStruct((N, W128), jnp.float32),
        ],
    )(h, dis, Wcat, b1)


def _edge_mlp_call(pA, pB, ea, W2, b2, g1, be1, g2, be2):
    BE = 2048

    def body(pA_ref, pB_ref, ea_ref, W2_ref, b2_ref, g1_ref, be1_ref, g2_ref,
             be2_ref, o_ref):
        i = pl.program_id(0)
        a = pA_ref[...]
        pre = a[:, :H] + pB_ref[:, :H]
        ef1 = _lrelu(_ln(pre, g1_ref[...], be1_ref[...]), 0.2)
        z = jnp.dot(ef1, W2_ref[...], preferred_element_type=jnp.float32,
                    precision=_PREC) + b2_ref[...]
        ef2 = _lrelu(_ln(z, g2_ref[...], be2_ref[...]), 0.2)
        ef2 = ef2 * jnp.reshape(ea_ref[pl.ds(i * BE, BE)], (BE, 1))
        o_ref[...] = jnp.concatenate([ef2, a[:, H:]], axis=1)

    return pl.pallas_call(
        body,
        grid=(EPAD // BE,),
        in_specs=[
            pl.BlockSpec((BE, W128), lambda i: (i, 0)),
            pl.BlockSpec((BE, W128), lambda i: (i, 0)),
            pl.BlockSpec((EPAD,), lambda i: (0,)),
            pl.BlockSpec((H, H), lambda i: (0, 0)),
            pl.BlockSpec((H,), lambda i: (0,)),
            pl.BlockSpec((H,), lambda i: (0,)),
            pl.BlockSpec((H,), lambda i: (0,)),
            pl.BlockSpec((H,), lambda i: (0,)),
            pl.BlockSpec((H,), lambda i: (0,)),
        ],
        out_specs=pl.BlockSpec((BE, W128), lambda i: (i, 0)),
        out_shape=jax.ShapeDtypeStruct((EPAD, W128), jnp.float32),
    )(pA, pB, ea, W2, b2, g1, be1, g2, be2)


def _combine_call(Sp, AG, dis, bg, ng, nbe, h0):
    BN = 2000
    residual = h0 is not None

    def body(sp_ref, ag_ref, dis_ref, bg_ref, ng_ref, nbe_ref, *rest):
        if residual:
            h0_ref, o_ref = rest
        else:
            (o_ref,) = rest
        t = sp_ref[0] + sp_ref[1]
        d2 = dis_ref[...]
        x1 = (t[:, H:] + ag_ref[:, H:]) * d2 + bg_ref[...]
        h = _lrelu(_ln(x1 + t[:, :H], ng_ref[...], nbe_ref[...]), 0.01)
        if residual:
            h = h + h0_ref[...]
        o_ref[...] = h

    in_specs = [
        pl.BlockSpec((NC, BN, W128), lambda i: (0, i, 0)),
        pl.BlockSpec((BN, W128), lambda i: (i, 0)),
        pl.BlockSpec((BN, 1), lambda i: (i, 0)),
        pl.BlockSpec((H,), lambda i: (0,)),
        pl.BlockSpec((H,), lambda i: (0,)),
        pl.BlockSpec((H,), lambda i: (0,)),
    ]
    args = [Sp, AG, dis, bg, ng, nbe]
    if residual:
        in_specs.append(pl.BlockSpec((BN, H), lambda i: (i, 0)))
        args.append(h0)
    return pl.pallas_call(
        body,
        grid=(N // BN,),
        in_specs=in_specs,
        out_specs=pl.BlockSpec((BN, H), lambda i: (i, 0)),
        out_shape=jax.ShapeDtypeStruct((N, H), jnp.float32),
    )(*args)


def _readout_call(h, tf2d, p):
    def body(h_ref, tf_ref, p1W, p1b, p2W, p2b, tW, tb, tg, tbe,
             v1W, v1b, vg, vbe, v2W, v2b, a1W, a1b, ag, abe, a2W, a2b,
             o_ref):
        hh = h_ref[...]
        w = _lrelu(jnp.dot(hh, p1W[...], preferred_element_type=jnp.float32,
                           precision=_PREC) + p1b[...], 0.01)
        s = jnp.dot(w, p2W[...], preferred_element_type=jnp.float32,
                    precision=_PREC) + p2b[...]
        m = jnp.max(s)
        e = jnp.exp(s - m)
        wn = e / jnp.sum(e)
        gf = lax.dot_general(wn, hh, (((0,), (0,)), ((), ())),
                             preferred_element_type=jnp.float32,
                             precision=_PREC)
        te = _lrelu(_ln(jnp.dot(tf_ref[...], tW[...],
                                preferred_element_type=jnp.float32,
                                precision=_PREC) + tb[...],
                        tg[...], tbe[...]), 0.01)
        cf = jnp.concatenate([gf, te], axis=1)
        v = jnp.dot(_lrelu(_ln(jnp.dot(cf, v1W[...],
                                       preferred_element_type=jnp.float32,
                                       precision=_PREC) + v1b[...],
                               vg[...], vbe[...]), 0.01), v2W[...],
                    preferred_element_type=jnp.float32,
                    precision=_PREC) + v2b[...]
        a = jnp.dot(_lrelu(_ln(jnp.dot(cf, a1W[...],
                                       preferred_element_type=jnp.float32,
                                       precision=_PREC) + a1b[...],
                               ag[...], abe[...]), 0.01), a2W[...],
                    preferred_element_type=jnp.float32,
                    precision=_PREC) + a2b[...]
        o_ref[...] = v + (a - jnp.mean(a, axis=-1, keepdims=True))

    return pl.pallas_call(
        body,
        out_shape=jax.ShapeDtypeStruct((1, OUT), jnp.float32),
    )(h, tf2d, p["p1_W"], p["p1_b"], p["p2_W"], p["p2_b"],
      p["t_W"], p["t_b"], p["t_g"], p["t_be"],
      p["v1_W"], p["v1_b"], p["v_g"], p["v_be"], p["v2_W"], p["v2_b"],
      p["a1_W"], p["a1_b"], p["a_g"], p["a_be"], p["a2_W"], p["a2_b"])


def _sc_mesh():
    return plsc.VectorSubcoreMesh(core_axis_name="c", subcore_axis_name="s")


def _zero_acc(z_hbm, acc, s):
    pltpu.sync_copy(z_hbm, acc.at[pl.ds(s * NPS, NPS)])

    @pl.when(s == NS - 1)
    def _():
        pltpu.sync_copy(z_hbm.at[pl.ds(0, TAIL)], acc.at[pl.ds(TOFF, TAIL)])


def _dump_acc(acc, out_hbm, c, s):
    pltpu.sync_copy(acc.at[pl.ds(s * NPS, NPS)],
                    out_hbm.at[c, pl.ds(s * NPS, NPS)])

    @pl.when(s == NS - 1)
    def _():
        pltpu.sync_copy(acc.at[pl.ds(TOFF, TAIL)],
                        out_hbm.at[c, pl.ds(TOFF, TAIL)])


def _sc_deg_call(dsts1d, ones128, z128):
    @functools.partial(
        pl.kernel,
        mesh=_sc_mesh(),
        out_type=jax.ShapeDtypeStruct((NC, N, W128), jnp.float32),
        scratch_types=[
            pltpu.VMEM((C,), jnp.int32),
            pltpu.VMEM((C, W128), jnp.float32),
            pltpu.VMEM_SHARED((ACCR, W128), jnp.float32),
        ],
    )
    def k(dst_hbm, ones_hbm, z_hbm, out_hbm, idx1, ones_v, acc):
        c = lax.axis_index("c")
        s = lax.axis_index("s")
        wid = s * NC + c
        pltpu.sync_copy(ones_hbm, ones_v)
        _zero_acc(z_hbm, acc, s)
        plsc.subcore_barrier()

        def chunk(j, carry):
            pltpu.sync_copy(dst_hbm.at[pl.ds(wid * EPWP + j * C, C)], idx1)
            pltpu.sync_copy(ones_v, acc.at[idx1], add=True)
            return carry

        lax.fori_loop(0, NCHP, chunk, 0)
        plsc.subcore_barrier()
        _dump_acc(acc, out_hbm, c, s)

    return k(dsts1d, ones128, z128)


def _sc_gather(AG, BB, srcg1d, dstg1d):
    @functools.partial(
        pl.kernel,
        mesh=_sc_mesh(),
        out_type=[
            jax.ShapeDtypeStruct((EPAD, W128), jnp.float32),
            jax.ShapeDtypeStruct((EPAD, W128), jnp.float32),
        ],
        scratch_types=[
            pltpu.VMEM((C,), jnp.int32),
            pltpu.VMEM((C,), jnp.int32),
            pltpu.VMEM((C,), jnp.int32),
            pltpu.VMEM((C,), jnp.int32),
            pltpu.VMEM((C, W128), jnp.float32),
            pltpu.VMEM((C, W128), jnp.float32),
            pltpu.VMEM((C, W128), jnp.float32),
            pltpu.VMEM((C, W128), jnp.float32),
        ] + [pltpu.SemaphoreType.DMA] * 12,
    )
    def k(AG_hbm, BB_hbm, src_hbm, dstg_hbm, pA_hbm, pB_hbm,
          is0, id0, is1, id1, bA0, bB0, bA1, bB1,
          si0, sj0, si1, sj1, sg0, sh0, sg1, sh1, sw0, sx0, sw1, sx1):
        c = lax.axis_index("c")
        s = lax.axis_index("s")
        wid = s * NC + c
        wbase = wid * EPWP

        pltpu.async_copy(src_hbm.at[pl.ds(wbase, C)], is0, si0)
        pltpu.async_copy(dstg_hbm.at[pl.ds(wbase, C)], id0, sj0)

        def one(j, isx, idx, bA, bB, isn, idn, si, sj, sg, sh, sw, sx,
                sin, sjn):
            base = wbase + j * C
            pltpu.make_async_copy(src_hbm.at[pl.ds(base, C)], isx, si).wait()
            pltpu.make_async_copy(dstg_hbm.at[pl.ds(base, C)], idx, sj).wait()

            @pl.when(j >= 2)
            def _():
                pbase = wbase + (j - 2) * C
                pltpu.make_async_copy(bA, pA_hbm.at[pl.ds(pbase, C)],
                                      sw).wait()
                pltpu.make_async_copy(bB, pB_hbm.at[pl.ds(pbase, C)],
                                      sx).wait()

            ha = pltpu.async_copy(AG_hbm.at[isx], bA, sg)
            hb = pltpu.async_copy(BB_hbm.at[idx], bB, sh)

            @pl.when(j + 1 < NCHP)
            def _():
                nbase = wbase + (j + 1) * C
                pltpu.async_copy(src_hbm.at[pl.ds(nbase, C)], isn, sin)
                pltpu.async_copy(dstg_hbm.at[pl.ds(nbase, C)], idn, sjn)

            ha.wait()
            hb.wait()
            pltpu.async_copy(bA, pA_hbm.at[pl.ds(base, C)], sw)
            pltpu.async_copy(bB, pB_hbm.at[pl.ds(base, C)], sx)

        def pair(i, carry):
            j = i * 2
            one(j, is0, id0, bA0, bB0, is1, id1,
                si0, sj0, sg0, sh0, sw0, sx0, si1, sj1)
            one(j + 1, is1, id1, bA1, bB1, is0, id0,
                si1, sj1, sg1, sh1, sw1, sx1, si0, sj0)
            return carry

        lax.fori_loop(0, NCHP // 2, pair, 0)

        pltpu.make_async_copy(bA0, pA_hbm.at[pl.ds(wbase + (NCHP - 2) * C, C)],
                              sw0).wait()
        pltpu.make_async_copy(bB0, pB_hbm.at[pl.ds(wbase + (NCHP - 2) * C, C)],
                              sx0).wait()
        pltpu.make_async_copy(bA1, pA_hbm.at[pl.ds(wbase + (NCHP - 1) * C, C)],
                              sw1).wait()
        pltpu.make_async_copy(bB1, pB_hbm.at[pl.ds(wbase + (NCHP - 1) * C, C)],
                              sx1).wait()

    return k(AG, BB, srcg1d, dstg1d)


def _sc_scatter(msg2, dsts1d, z128):
    @functools.partial(
        pl.kernel,
        mesh=_sc_mesh(),
        out_type=jax.ShapeDtypeStruct((NC, N, W128), jnp.float32),
        scratch_types=[
            pltpu.VMEM((C,), jnp.int32),
            pltpu.VMEM((C, W128), jnp.float32),
            pltpu.VMEM_SHARED((ACCR, W128), jnp.float32),
        ],
    )
    def k(ef_hbm, dst_hbm, z_hbm, s2_hbm, idx_d, bufE, acc):
        c = lax.axis_index("c")
        s = lax.axis_index("s")
        wid = s * NC + c
        _zero_acc(z_hbm, acc, s)
        plsc.subcore_barrier()

        def chunk(j, carry):
            base = wid * EPWP + j * C
            pltpu.sync_copy(dst_hbm.at[pl.ds(base, C)], idx_d)
            pltpu.sync_copy(ef_hbm.at[pl.ds(base, C)], bufE)
            pltpu.sync_copy(bufE, acc.at[idx_d], add=True)
            return carry

        lax.fori_loop(0, NCHP, chunk, 0)
        plsc.subcore_barrier()
        _dump_acc(acc, s2_hbm, c, s)

    return k(msg2, dsts1d, z128)


def _pad_per_worker(v, fill):
    vw = v.reshape(NW, EPW)
    return jnp.pad(vw, ((0, 0), (0, EPWP - EPW)),
                   constant_values=fill).reshape(-1)


def kernel(x, edge_index, edge_attr, task_features, params):
    p = params
    src = edge_index[0]
    dst = edge_index[1]
    srcg1d = _pad_per_worker(src, 0)
    dstg1d = _pad_per_worker(dst, 0)
    dsts1d = _pad_per_worker(dst, N)
    ea_p = _pad_per_worker(edge_attr, 0.0)
    z128 = jnp.zeros((NPS, W128), jnp.float32)
    ones128 = jnp.ones((C, W128), jnp.float32)

    degp = _sc_deg_call(dsts1d, ones128, z128)
    dis = _dis_call(degp)
    h = _encode_call(x, p["ne_W"], p["ne_b"], p["ne_g"], p["ne_be"])
    h0 = h
    for i in range(L):
        W1 = p["e1_W%d" % i]
        Wcat = jnp.concatenate([W1[:H], p["gcn_W%d" % i], W1[H:]], axis=1)
        AG, BB = _layer_a_call(h, dis, Wcat, p["e1_b%d" % i])
        pA, pB = _sc_gather(AG, BB, srcg1d, dstg1d)
        msg2 = _edge_mlp_call(pA, pB, ea_p, p["e2_W%d" % i], p["e2_b%d" % i],
                              p["e1_g%d" % i], p["e1_be%d" % i],
                              p["e2_g%d" % i], p["e2_be%d" % i])
        Sp = _sc_scatter(msg2, dsts1d, z128)
        h = _combine_call(Sp, AG, dis, p["gcn_b%d" % i],
                          p["n_g%d" % i], p["n_be%d" % i],
                          h0 if i > 0 else None)
        if i > 0:
            h0 = h
    out = _readout_call(h, task_features.reshape(1, 4), p)
    return out[0]


# speedup vs baseline: 4.9531x; 1.0736x over previous
import functools

import jax
import jax.numpy as jnp
from jax import lax
from jax.experimental import pallas as pl
from jax.experimental.pallas import tpu as pltpu
from jax.experimental.pallas import tpu_sc as plsc

N = 10000
E = 320000
DIN = 128
H = 64
OUT = 10
L = 3
W128 = 2 * H

NC = 2
NS = 16
NW = NC * NS
EPW = E // NW
C = 128
NCHP = 80
EPWP = NCHP * C
EPAD = NW * EPWP
ACCR = N + 16
NPS = 624
TAIL = N - NPS * NS
TOFF = NPS * NS

_PREC = lax.Precision.HIGHEST


def _lrelu(x, s):
    return jnp.where(x >= 0, x, s * x)


def _ln(x, g, b, eps=1e-5):
    m = jnp.mean(x, axis=-1, keepdims=True)
    v = jnp.mean(jnp.square(x - m), axis=-1, keepdims=True)
    return (x - m) * lax.rsqrt(v + eps) * g + b


def _encode_call(x, W, b, g, be):
    BN = 2000

    def body(x_ref, W_ref, b_ref, g_ref, be_ref, o_ref):
        h = jnp.dot(x_ref[...], W_ref[...], preferred_element_type=jnp.float32,
                    precision=_PREC) + b_ref[...]
        o_ref[...] = _lrelu(_ln(h, g_ref[...], be_ref[...]), 0.01)

    return pl.pallas_call(
        body,
        grid=(N // BN,),
        in_specs=[
            pl.BlockSpec((BN, DIN), lambda i: (i, 0)),
            pl.BlockSpec((DIN, H), lambda i: (0, 0)),
            pl.BlockSpec((H,), lambda i: (0,)),
            pl.BlockSpec((H,), lambda i: (0,)),
            pl.BlockSpec((H,), lambda i: (0,)),
        ],
        out_specs=pl.BlockSpec((BN, H), lambda i: (i, 0)),
        out_shape=jax.ShapeDtypeStruct((N, H), jnp.float32),
    )(x, W, b, g, be)


def _dis_call(degp):
    BN = 2000

    def body(d_ref, o_ref):
        t = d_ref[0] + d_ref[1]
        cnt = t[:, 0:1]
        o_ref[...] = lax.rsqrt(cnt + 1.0)

    return pl.pallas_call(
        body,
        grid=(N // BN,),
        in_specs=[pl.BlockSpec((NC, BN, W128), lambda i: (0, i, 0))],
        out_specs=pl.BlockSpec((BN, 1), lambda i: (i, 0)),
        out_shape=jax.ShapeDtypeStruct((N, 1), jnp.float32),
    )(degp)


def _layer_a_call(h, dis, Wcat, b1):
    BN = 2000

    def body(h_ref, dis_ref, W_ref, b1_ref, AG_ref, BB_ref):
        t = jnp.dot(h_ref[...], W_ref[...], preferred_element_type=jnp.float32,
                    precision=_PREC)
        d2 = dis_ref[...]
        AG_ref[...] = jnp.concatenate([t[:, :H], t[:, H:2 * H] * d2], axis=1)
        BB_ref[...] = jnp.concatenate(
            [t[:, 2 * H:] + b1_ref[...], jnp.zeros((BN, H), jnp.float32)],
            axis=1)

    return pl.pallas_call(
        body,
        grid=(N // BN,),
        in_specs=[
            pl.BlockSpec((BN, H), lambda i: (i, 0)),
            pl.BlockSpec((BN, 1), lambda i: (i, 0)),
            pl.BlockSpec((H, 3 * H), lambda i: (0, 0)),
            pl.BlockSpec((H,), lambda i: (0,)),
        ],
        out_specs=[
            pl.BlockSpec((BN, W128), lambda i: (i, 0)),
            pl.BlockSpec((BN, W128), lambda i: (i, 0)),
        ],
        out_shape=[
            jax.ShapeDtypeStruct((N, W128), jnp.float32),
            jax.ShapeDtypeStruct((N, W128), jnp.float32),
        ],
    )(h, dis, Wcat, b1)


def _edge_mlp_call(pA, pB, ea, W2, b2, g1, be1, g2, be2):
    BE = 2048

    def body(pA_ref, pB_ref, ea_ref, W2_ref, b2_ref, g1_ref, be1_ref, g2_ref,
             be2_ref, o_ref):
        i = pl.program_id(0)
        a = pA_ref[...]
        pre = a[:, :H] + pB_ref[:, :H]
        ef1 = _lrelu(_ln(pre, g1_ref[...], be1_ref[...]), 0.2)
        z = jnp.dot(ef1, W2_ref[...], preferred_element_type=jnp.float32,
                    precision=_PREC) + b2_ref[...]
        ef2 = _lrelu(_ln(z, g2_ref[...], be2_ref[...]), 0.2)
        ef2 = ef2 * jnp.reshape(ea_ref[pl.ds(i * BE, BE)], (BE, 1))
        o_ref[...] = jnp.concatenate([ef2, a[:, H:]], axis=1)

    return pl.pallas_call(
        body,
        grid=(EPAD // BE,),
        in_specs=[
            pl.BlockSpec((BE, W128), lambda i: (i, 0)),
            pl.BlockSpec((BE, W128), lambda i: (i, 0)),
            pl.BlockSpec((EPAD,), lambda i: (0,)),
            pl.BlockSpec((H, H), lambda i: (0, 0)),
            pl.BlockSpec((H,), lambda i: (0,)),
            pl.BlockSpec((H,), lambda i: (0,)),
            pl.BlockSpec((H,), lambda i: (0,)),
            pl.BlockSpec((H,), lambda i: (0,)),
            pl.BlockSpec((H,), lambda i: (0,)),
        ],
        out_specs=pl.BlockSpec((BE, W128), lambda i: (i, 0)),
        out_shape=jax.ShapeDtypeStruct((EPAD, W128), jnp.float32),
    )(pA, pB, ea, W2, b2, g1, be1, g2, be2)


def _combine_call(Sp, AG, dis, bg, ng, nbe, h0):
    BN = 2000
    residual = h0 is not None

    def body(sp_ref, ag_ref, dis_ref, bg_ref, ng_ref, nbe_ref, *rest):
        if residual:
            h0_ref, o_ref = rest
        else:
            (o_ref,) = rest
        t = sp_ref[0] + sp_ref[1]
        d2 = dis_ref[...]
        x1 = (t[:, H:] + ag_ref[:, H:]) * d2 + bg_ref[...]
        h = _lrelu(_ln(x1 + t[:, :H], ng_ref[...], nbe_ref[...]), 0.01)
        if residual:
            h = h + h0_ref[...]
        o_ref[...] = h

    in_specs = [
        pl.BlockSpec((NC, BN, W128), lambda i: (0, i, 0)),
        pl.BlockSpec((BN, W128), lambda i: (i, 0)),
        pl.BlockSpec((BN, 1), lambda i: (i, 0)),
        pl.BlockSpec((H,), lambda i: (0,)),
        pl.BlockSpec((H,), lambda i: (0,)),
        pl.BlockSpec((H,), lambda i: (0,)),
    ]
    args = [Sp, AG, dis, bg, ng, nbe]
    if residual:
        in_specs.append(pl.BlockSpec((BN, H), lambda i: (i, 0)))
        args.append(h0)
    return pl.pallas_call(
        body,
        grid=(N // BN,),
        in_specs=in_specs,
        out_specs=pl.BlockSpec((BN, H), lambda i: (i, 0)),
        out_shape=jax.ShapeDtypeStruct((N, H), jnp.float32),
    )(*args)


def _readout_call(h, tf2d, p):
    def body(h_ref, tf_ref, p1W, p1b, p2W, p2b, tW, tb, tg, tbe,
             v1W, v1b, vg, vbe, v2W, v2b, a1W, a1b, ag, abe, a2W, a2b,
             o_ref):
        hh = h_ref[...]
        w = _lrelu(jnp.dot(hh, p1W[...], preferred_element_type=jnp.float32,
                           precision=_PREC) + p1b[...], 0.01)
        s = jnp.dot(w, p2W[...], preferred_element_type=jnp.float32,
                    precision=_PREC) + p2b[...]
        m = jnp.max(s)
        e = jnp.exp(s - m)
        wn = e / jnp.sum(e)
        gf = lax.dot_general(wn, hh, (((0,), (0,)), ((), ())),
                             preferred_element_type=jnp.float32,
                             precision=_PREC)
        te = _lrelu(_ln(jnp.dot(tf_ref[...], tW[...],
                                preferred_element_type=jnp.float32,
                                precision=_PREC) + tb[...],
                        tg[...], tbe[...]), 0.01)
        cf = jnp.concatenate([gf, te], axis=1)
        v = jnp.dot(_lrelu(_ln(jnp.dot(cf, v1W[...],
                                       preferred_element_type=jnp.float32,
                                       precision=_PREC) + v1b[...],
                               vg[...], vbe[...]), 0.01), v2W[...],
                    preferred_element_type=jnp.float32,
                    precision=_PREC) + v2b[...]
        a = jnp.dot(_lrelu(_ln(jnp.dot(cf, a1W[...],
                                       preferred_element_type=jnp.float32,
                                       precision=_PREC) + a1b[...],
                               ag[...], abe[...]), 0.01), a2W[...],
                    preferred_element_type=jnp.float32,
                    precision=_PREC) + a2b[...]
        o_ref[...] = v + (a - jnp.mean(a, axis=-1, keepdims=True))

    return pl.pallas_call(
        body,
        out_shape=jax.ShapeDtypeStruct((1, OUT), jnp.float32),
    )(h, tf2d, p["p1_W"], p["p1_b"], p["p2_W"], p["p2_b"],
      p["t_W"], p["t_b"], p["t_g"], p["t_be"],
      p["v1_W"], p["v1_b"], p["v_g"], p["v_be"], p["v2_W"], p["v2_b"],
      p["a1_W"], p["a1_b"], p["a_g"], p["a_be"], p["a2_W"], p["a2_b"])


def _sc_mesh():
    return plsc.VectorSubcoreMesh(core_axis_name="c", subcore_axis_name="s")


def _zero_acc(z_hbm, acc, s):
    pltpu.sync_copy(z_hbm, acc.at[pl.ds(s * NPS, NPS)])

    @pl.when(s == NS - 1)
    def _():
        pltpu.sync_copy(z_hbm.at[pl.ds(0, TAIL)], acc.at[pl.ds(TOFF, TAIL)])


def _dump_acc(acc, out_hbm, c, s):
    pltpu.sync_copy(acc.at[pl.ds(s * NPS, NPS)],
                    out_hbm.at[c, pl.ds(s * NPS, NPS)])

    @pl.when(s == NS - 1)
    def _():
        pltpu.sync_copy(acc.at[pl.ds(TOFF, TAIL)],
                        out_hbm.at[c, pl.ds(TOFF, TAIL)])


def _sc_deg_call(dsts1d, ones128, z128):
    @functools.partial(
        pl.kernel,
        mesh=_sc_mesh(),
        out_type=jax.ShapeDtypeStruct((NC, N, W128), jnp.float32),
        scratch_types=[
            pltpu.VMEM((C,), jnp.int32),
            pltpu.VMEM((C, W128), jnp.float32),
            pltpu.VMEM_SHARED((ACCR, W128), jnp.float32),
        ],
    )
    def k(dst_hbm, ones_hbm, z_hbm, out_hbm, idx1, ones_v, acc):
        c = lax.axis_index("c")
        s = lax.axis_index("s")
        wid = s * NC + c
        pltpu.sync_copy(ones_hbm, ones_v)
        _zero_acc(z_hbm, acc, s)
        plsc.subcore_barrier()

        def chunk(j, carry):
            pltpu.sync_copy(dst_hbm.at[pl.ds(wid * EPWP + j * C, C)], idx1)
            pltpu.sync_copy(ones_v, acc.at[idx1], add=True)
            return carry

        lax.fori_loop(0, NCHP, chunk, 0)
        plsc.subcore_barrier()
        _dump_acc(acc, out_hbm, c, s)

    return k(dsts1d, ones128, z128)


def _sc_gather(AG, BB, srcg1d, dstg1d):
    @functools.partial(
        pl.kernel,
        mesh=_sc_mesh(),
        out_type=[
            jax.ShapeDtypeStruct((EPAD, W128), jnp.float32),
            jax.ShapeDtypeStruct((EPAD, W128), jnp.float32),
        ],
        scratch_types=[
            pltpu.VMEM((C,), jnp.int32),
            pltpu.VMEM((C,), jnp.int32),
            pltpu.VMEM((C,), jnp.int32),
            pltpu.VMEM((C,), jnp.int32),
            pltpu.VMEM((C, W128), jnp.float32),
            pltpu.VMEM((C, W128), jnp.float32),
            pltpu.VMEM((C, W128), jnp.float32),
            pltpu.VMEM((C, W128), jnp.float32),
        ] + [pltpu.SemaphoreType.DMA] * 12,
    )
    def k(AG_hbm, BB_hbm, src_hbm, dstg_hbm, pA_hbm, pB_hbm,
          is0, id0, is1, id1, bA0, bB0, bA1, bB1,
          si0, sj0, si1, sj1, sg0, sh0, sg1, sh1, sw0, sx0, sw1, sx1):
        c = lax.axis_index("c")
        s = lax.axis_index("s")
        wid = s * NC + c
        wbase = wid * EPWP

        pltpu.sync_copy(src_hbm.at[pl.ds(wbase, C)], is0)
        pltpu.sync_copy(dstg_hbm.at[pl.ds(wbase, C)], id0)
        pltpu.async_copy(AG_hbm.at[is0], bA0, sg0)
        pltpu.async_copy(BB_hbm.at[id0], bB0, sh0)
        pltpu.async_copy(src_hbm.at[pl.ds(wbase + C, C)], is1, si1)
        pltpu.async_copy(dstg_hbm.at[pl.ds(wbase + C, C)], id1, sj1)

        def one(j, isx, idx, bA, bB, isq, idq, bAq, bBq,
                si, sj, sg, sh, sw, sx, sgq, shq, swq, sxq, siq, sjq):
            base = wbase + j * C
            pltpu.make_async_copy(src_hbm.at[pl.ds(base, C)], isx, si).wait()
            pltpu.make_async_copy(dstg_hbm.at[pl.ds(base, C)], idx, sj).wait()

            @pl.when(j >= 2)
            def _():
                pbase = wbase + (j - 2) * C
                pltpu.make_async_copy(bA, pA_hbm.at[pl.ds(pbase, C)],
                                      sw).wait()
                pltpu.make_async_copy(bB, pB_hbm.at[pl.ds(pbase, C)],
                                      sx).wait()

            pltpu.async_copy(AG_hbm.at[isx], bA, sg)
            pltpu.async_copy(BB_hbm.at[idx], bB, sh)

            qbase = wbase + (j - 1) * C
            pltpu.make_async_copy(AG_hbm.at[isq], bAq, sgq).wait()
            pltpu.make_async_copy(BB_hbm.at[idq], bBq, shq).wait()
            pltpu.async_copy(bAq, pA_hbm.at[pl.ds(qbase, C)], swq)
            pltpu.async_copy(bBq, pB_hbm.at[pl.ds(qbase, C)], sxq)

            @pl.when(j + 1 < NCHP)
            def _():
                nbase = wbase + (j + 1) * C
                pltpu.async_copy(src_hbm.at[pl.ds(nbase, C)], isq, siq)
                pltpu.async_copy(dstg_hbm.at[pl.ds(nbase, C)], idq, sjq)

        def body(j, carry):
            @pl.when(j % 2 == 1)
            def _():
                one(j, is1, id1, bA1, bB1, is0, id0, bA0, bB0,
                    si1, sj1, sg1, sh1, sw1, sx1, sg0, sh0, sw0, sx0,
                    si0, sj0)

            @pl.when(j % 2 == 0)
            def _():
                one(j, is0, id0, bA0, bB0, is1, id1, bA1, bB1,
                    si0, sj0, sg0, sh0, sw0, sx0, sg1, sh1, sw1, sx1,
                    si1, sj1)

            return carry

        lax.fori_loop(1, NCHP, body, 0)

        lbase = wbase + (NCHP - 1) * C
        pltpu.make_async_copy(AG_hbm.at[is1], bA1, sg1).wait()
        pltpu.make_async_copy(BB_hbm.at[id1], bB1, sh1).wait()
        pltpu.async_copy(bA1, pA_hbm.at[pl.ds(lbase, C)], sw1)
        pltpu.async_copy(bB1, pB_hbm.at[pl.ds(lbase, C)], sx1)
        pbase = wbase + (NCHP - 2) * C
        pltpu.make_async_copy(bA0, pA_hbm.at[pl.ds(pbase, C)], sw0).wait()
        pltpu.make_async_copy(bB0, pB_hbm.at[pl.ds(pbase, C)], sx0).wait()
        pltpu.make_async_copy(bA1, pA_hbm.at[pl.ds(lbase, C)], sw1).wait()
        pltpu.make_async_copy(bB1, pB_hbm.at[pl.ds(lbase, C)], sx1).wait()

    return k(AG, BB, srcg1d, dstg1d)


def _sc_scatter(msg2, dsts1d, z128):
    @functools.partial(
        pl.kernel,
        mesh=_sc_mesh(),
        out_type=jax.ShapeDtypeStruct((NC, N, W128), jnp.float32),
        scratch_types=[
            pltpu.VMEM((C,), jnp.int32),
            pltpu.VMEM((C,), jnp.int32),
            pltpu.VMEM((C, W128), jnp.float32),
            pltpu.VMEM((C, W128), jnp.float32),
            pltpu.SemaphoreType.DMA,
            pltpu.SemaphoreType.DMA,
            pltpu.SemaphoreType.DMA,
            pltpu.SemaphoreType.DMA,
            pltpu.VMEM_SHARED((ACCR, W128), jnp.float32),
        ],
    )
    def k(ef_hbm, dst_hbm, z_hbm, s2_hbm, id0, id1, bE0, bE1,
          si0, sj0, si1, sj1, acc):
        c = lax.axis_index("c")
        s = lax.axis_index("s")
        wid = s * NC + c
        wbase = wid * EPWP
        pltpu.async_copy(dst_hbm.at[pl.ds(wbase, C)], id0, si0)
        pltpu.async_copy(ef_hbm.at[pl.ds(wbase, C)], bE0, sj0)
        _zero_acc(z_hbm, acc, s)
        plsc.subcore_barrier()

        def one(j, idx, bE, idq, bEq, si, sj, siq, sjq):
            base = wbase + j * C
            pltpu.make_async_copy(dst_hbm.at[pl.ds(base, C)], idx, si).wait()
            pltpu.make_async_copy(ef_hbm.at[pl.ds(base, C)], bE, sj).wait()

            @pl.when(j + 1 < NCHP)
            def _():
                nbase = wbase + (j + 1) * C
                pltpu.async_copy(dst_hbm.at[pl.ds(nbase, C)], idq, siq)
                pltpu.async_copy(ef_hbm.at[pl.ds(nbase, C)], bEq, sjq)

            pltpu.sync_copy(bE, acc.at[idx], add=True)

        def body(j, carry):
            @pl.when(j % 2 == 0)
            def _():
                one(j, id0, bE0, id1, bE1, si0, sj0, si1, sj1)

            @pl.when(j % 2 == 1)
            def _():
                one(j, id1, bE1, id0, bE0, si1, sj1, si0, sj0)

            return carry

        lax.fori_loop(0, NCHP, body, 0)
        plsc.subcore_barrier()
        _dump_acc(acc, s2_hbm, c, s)

    return k(msg2, dsts1d, z128)


def _pad_per_worker(v, fill):
    vw = v.reshape(NW, EPW)
    return jnp.pad(vw, ((0, 0), (0, EPWP - EPW)),
                   constant_values=fill).reshape(-1)


def kernel(x, edge_index, edge_attr, task_features, params):
    p = params
    src = edge_index[0]
    dst = edge_index[1]
    srcg1d = _pad_per_worker(src, 0)
    dstg1d = _pad_per_worker(dst, 0)
    dsts1d = _pad_per_worker(dst, N)
    ea_p = _pad_per_worker(edge_attr, 0.0)
    z128 = jnp.zeros((NPS, W128), jnp.float32)
    ones128 = jnp.ones((C, W128), jnp.float32)

    degp = _sc_deg_call(dsts1d, ones128, z128)
    dis = _dis_call(degp)
    h = _encode_call(x, p["ne_W"], p["ne_b"], p["ne_g"], p["ne_be"])
    h0 = h
    for i in range(L):
        W1 = p["e1_W%d" % i]
        Wcat = jnp.concatenate([W1[:H], p["gcn_W%d" % i], W1[H:]], axis=1)
        AG, BB = _layer_a_call(h, dis, Wcat, p["e1_b%d" % i])
        pA, pB = _sc_gather(AG, BB, srcg1d, dstg1d)
        msg2 = _edge_mlp_call(pA, pB, ea_p, p["e2_W%d" % i], p["e2_b%d" % i],
                              p["e1_g%d" % i], p["e1_be%d" % i],
                              p["e2_g%d" % i], p["e2_be%d" % i])
        Sp = _sc_scatter(msg2, dsts1d, z128)
        h = _combine_call(Sp, AG, dis, p["gcn_b%d" % i],
                          p["n_g%d" % i], p["n_be%d" % i],
                          h0 if i > 0 else None)
        if i > 0:
            h0 = h
    out = _readout_call(h, task_features.reshape(1, 4), p)
    return out[0]
